# baseline (speedup 1.0000x reference)
import functools

import jax
import jax.numpy as jnp
from jax import lax
from jax.experimental import pallas as pl
from jax.experimental.pallas import tpu as pltpu
from jax.experimental.pallas import tpu_sc as plsc

_N = 10000
_E = 320000
_D = 128

_NC = 2
_NS = 16
_NW = _NC * _NS
_L = 128
_P = 80
_E_PAD = _NW * _P * _L
_NB = 2
_SS = 4
_SL = _L // _SS
_CH = 8
_NCH = _P // _CH
_N_PAD = 10240
_ZR = _N_PAD // _NS

_mesh = plsc.VectorSubcoreMesh(core_axis_name="c", subcore_axis_name="s")


def _sc_degree(dst2d, ones2d, zerosD):
    @functools.partial(
        pl.kernel,
        out_type=jax.ShapeDtypeStruct((_NC, _N_PAD, _D), jnp.float32),
        mesh=_mesh,
        scratch_types=[
            pltpu.VMEM((_P, _L), jnp.int32),
            pltpu.VMEM((_L, _D), jnp.float32),
            pltpu.VMEM_SHARED((_N_PAD, _D), jnp.float32),
        ],
    )
    def k(dst_hbm, ones_hbm, z_hbm, out_hbm, dst_v, ones_v, acc):
        c = lax.axis_index("c")
        s = lax.axis_index("s")
        wid = c * _NS + s
        pltpu.sync_copy(z_hbm.at[pl.ds(s * _ZR, _ZR)], acc.at[pl.ds(s * _ZR, _ZR)])
        pltpu.sync_copy(dst_hbm.at[pl.ds(wid * _P, _P)], dst_v)
        pltpu.sync_copy(ones_hbm, ones_v)
        plsc.subcore_barrier()

        @pl.loop(0, _P)
        def _(j):
            pltpu.sync_copy(ones_v, acc.at[dst_v.at[j]], add=True)

        plsc.subcore_barrier()
        pltpu.sync_copy(acc.at[pl.ds(s * _ZR, _ZR)],
                        out_hbm.at[c].at[pl.ds(s * _ZR, _ZR)])

    return k(dst2d, ones2d, zerosD)


def _sc_scatter(h, src2d, dst2d, zeros_init):
    @functools.partial(
        pl.kernel,
        out_type=jax.ShapeDtypeStruct((_NC, _N_PAD, _D), jnp.float32),
        mesh=_mesh,
        scratch_types=[
            pltpu.VMEM((_CH * _L,), jnp.int32),
            pltpu.VMEM((_CH * _L,), jnp.int32),
            pltpu.VMEM((_CH, _L), jnp.int32),
            pltpu.VMEM((_CH, _L), jnp.int32),
            pltpu.VMEM((_L, _D), jnp.float32),
            pltpu.VMEM((_L, _D), jnp.float32),
            pltpu.VMEM_SHARED((_N_PAD, _D), jnp.float32),
            pltpu.SemaphoreType.DMA((2,)),
            pltpu.SemaphoreType.DMA((_NB,)),
        ],
    )
    def k(h_hbm, src_hbm, dst_hbm, z_hbm, out_hbm,
          src_a, src_b, dst_a, dst_b, rows_a, rows_b, acc, isem, gsem):
        srcb = (src_a, src_b)
        dstb = (dst_a, dst_b)
        rows = (rows_a, rows_b)
        c = lax.axis_index("c")
        s = lax.axis_index("s")
        wid = c * _NS + s
        row0 = wid * _P
        e0 = row0 * _L
        pltpu.sync_copy(z_hbm.at[pl.ds(s * _ZR, _ZR)], acc.at[pl.ds(s * _ZR, _ZR)])
        pltpu.async_copy(src_hbm.at[pl.ds(e0, _CH * _L)], src_a, isem.at[0])
        pltpu.async_copy(dst_hbm.at[pl.ds(row0, _CH)], dst_a, isem.at[0])
        plsc.subcore_barrier()

        def do_chunk(g, u):
            sv = srcb[u]
            dv = dstb[u]
            pltpu.make_async_copy(src_hbm.at[pl.ds(0, _CH * _L)], sv,
                                  isem.at[u]).wait()
            pltpu.make_async_copy(dst_hbm.at[pl.ds(0, _CH)], dv,
                                  isem.at[u]).wait()

            @pl.when(g + 1 < _NCH)
            def _():
                nxt = row0 + (g + 1) * _CH
                pltpu.async_copy(src_hbm.at[pl.ds(e0 + (g + 1) * _CH * _L,
                                                  _CH * _L)],
                                 srcb[1 - u], isem.at[1 - u])
                pltpu.async_copy(dst_hbm.at[pl.ds(nxt, _CH)], dstb[1 - u],
                                 isem.at[1 - u])

            def fire(i, b):
                for t in range(_SS):
                    pltpu.async_copy(
                        h_hbm.at[sv.at[pl.ds(i * _L + t * _SL, _SL)]],
                        rows[b].at[pl.ds(t * _SL, _SL)], gsem.at[b])

            for b in range(_NB):
                fire(b, b)
            for i in range(_CH):
                b = i % _NB
                pltpu.make_async_copy(h_hbm.at[pl.ds(0, _L)], rows[b],
                                      gsem.at[b]).wait()
                pltpu.sync_copy(rows[b], acc.at[dv.at[i]], add=True)
                if i + _NB < _CH:
                    fire(i + _NB, b)

        @pl.loop(0, _NCH // 2)
        def _(grp):
            do_chunk(grp * 2, 0)
            do_chunk(grp * 2 + 1, 1)

        plsc.subcore_barrier()
        pltpu.sync_copy(acc.at[pl.ds(s * _ZR, _ZR)],
                        out_hbm.at[c].at[pl.ds(s * _ZR, _ZR)])

    return k(h, src2d, dst2d, zeros_init)


def _tc_matmul(x, W):
    def body(x_ref, w_ref, o_ref):
        o_ref[...] = jnp.dot(x_ref[...], w_ref[...],
                             preferred_element_type=jnp.float32,
                             precision=lax.Precision.HIGHEST)

    return pl.pallas_call(
        body, out_shape=jax.ShapeDtypeStruct((_N, _D), jnp.float32))(x, W)


def _tc_prep(degp, h_raw):
    def body(dp_ref, h_ref, dinv_ref, hp_ref):
        deg = dp_ref[0, : _N, 0:1] + dp_ref[1, : _N, 0:1] + 1.0
        dinv = lax.rsqrt(deg)
        dinv_ref[...] = dinv
        hp_ref[...] = h_ref[...] * dinv

    return pl.pallas_call(
        body,
        out_shape=(jax.ShapeDtypeStruct((_N, 1), jnp.float32),
                   jax.ShapeDtypeStruct((_N, _D), jnp.float32)),
    )(degp, h_raw)


def _tc_post(p, hp, dinv, b, g, bt, W_next):
    def body(p_ref, hp_ref, dinv_ref, b_ref, g_ref, bt_ref, *rest):
        t = (p_ref[0, : _N] + p_ref[1, : _N] + hp_ref[...]) * dinv_ref[...] + b_ref[...]
        mu = jnp.mean(t, axis=0, keepdims=True)
        var = jnp.mean((t - mu) ** 2, axis=0, keepdims=True)
        y = jax.nn.relu(g_ref[...] * (t - mu) * lax.rsqrt(var + 1e-5)
                        + bt_ref[...])
        if W_next is None:
            rest[0][...] = y
        else:
            w_ref, o_ref = rest
            o_ref[...] = jnp.dot(y, w_ref[...],
                                 preferred_element_type=jnp.float32,
                                 precision=lax.Precision.HIGHEST) * dinv_ref[...]

    out_shape = jax.ShapeDtypeStruct((_N, _D), jnp.float32)
    args = (p, hp, dinv, b, g, bt)
    if W_next is not None:
        args = args + (W_next,)
    return pl.pallas_call(body, out_shape=out_shape)(*args)


def kernel(x, edge_index, W1, b1, g1, bt1, W2, b2, g2, bt2):
    src = edge_index[0]
    dst = edge_index[1]
    pad = _E_PAD - _E
    src1d = jnp.concatenate([src, jnp.zeros((pad,), jnp.int32)])
    dst2d = jnp.concatenate(
        [dst, jnp.full((pad,), _N, jnp.int32)]).reshape(_NW * _P, _L)
    ones2d = jnp.ones((_L, _D), jnp.float32)
    zerosD = jnp.zeros((_N_PAD, _D), jnp.float32)

    degp = _sc_degree(dst2d, ones2d, zerosD)
    h1_raw = _tc_matmul(x, W1)
    dinv, h1p = _tc_prep(degp, h1_raw)
    p1 = _sc_scatter(h1p, src1d, dst2d, zerosD)
    h2p = _tc_post(p1, h1p, dinv, b1, g1, bt1, W2)
    p2 = _sc_scatter(h2p, src1d, dst2d, zerosD)
    return _tc_post(p2, h2p, dinv, b2, g2, bt2, None)


# speedup vs baseline: 2.8417x; 2.8417x over previous
import functools

import jax
import jax.numpy as jnp
from jax import lax
from jax.experimental import pallas as pl
from jax.experimental.pallas import tpu as pltpu
from jax.experimental.pallas import tpu_sc as plsc

_N = 10000
_E = 320000
_D = 128

_NC = 2
_NS = 16
_NW = _NC * _NS
_L = 128
_P = 80
_E_PAD = _NW * _P * _L
_NB = 2
_CH = 8
_NCH = _P // _CH
_N_PAD = 10240
_ZR = _N_PAD // _NS

_mesh = plsc.VectorSubcoreMesh(core_axis_name="c", subcore_axis_name="s")


def _sc_degree(dst2d, ones2d, zerosD):
    @functools.partial(
        pl.kernel,
        out_type=jax.ShapeDtypeStruct((_NC, _N_PAD, _D), jnp.float32),
        mesh=_mesh,
        scratch_types=[
            pltpu.VMEM((_P, _L), jnp.int32),
            pltpu.VMEM((_L, _D), jnp.float32),
            pltpu.VMEM_SHARED((_N_PAD, _D), jnp.float32),
        ],
    )
    def k(dst_hbm, ones_hbm, z_hbm, out_hbm, dst_v, ones_v, acc):
        c = lax.axis_index("c")
        s = lax.axis_index("s")
        wid = c * _NS + s
        pltpu.sync_copy(z_hbm.at[pl.ds(s * _ZR, _ZR)], acc.at[pl.ds(s * _ZR, _ZR)])
        pltpu.sync_copy(dst_hbm.at[pl.ds(wid * _P, _P)], dst_v)
        pltpu.sync_copy(ones_hbm, ones_v)
        plsc.subcore_barrier()

        @pl.loop(0, _P)
        def _(j):
            pltpu.sync_copy(ones_v, acc.at[dst_v.at[j]], add=True)

        plsc.subcore_barrier()
        pltpu.sync_copy(acc.at[pl.ds(s * _ZR, _ZR)],
                        out_hbm.at[c].at[pl.ds(s * _ZR, _ZR)])

    return k(dst2d, ones2d, zerosD)


def _sc_scatter(h, src2d, dst2d, zeros_init):
    @functools.partial(
        pl.kernel,
        out_type=jax.ShapeDtypeStruct((_NC, _N_PAD, _D), jnp.float32),
        mesh=_mesh,
        scratch_types=[
            pltpu.VMEM((_CH * _L,), jnp.int32),
            pltpu.VMEM((_CH * _L,), jnp.int32),
            pltpu.VMEM((_CH, _L), jnp.int32),
            pltpu.VMEM((_CH, _L), jnp.int32),
            pltpu.VMEM((_L, _D), jnp.float32),
            pltpu.VMEM((_L, _D), jnp.float32),
            pltpu.VMEM_SHARED((_N_PAD, _D), jnp.float32),
            pltpu.SemaphoreType.DMA((2,)),
            pltpu.SemaphoreType.DMA((_NB,)),
        ],
    )
    def k(h_hbm, src_hbm, dst_hbm, z_hbm, out_hbm,
          src_a, src_b, dst_a, dst_b, rows_a, rows_b, acc, isem, gsem):
        srcb = (src_a, src_b)
        dstb = (dst_a, dst_b)
        rows = (rows_a, rows_b)
        c = lax.axis_index("c")
        s = lax.axis_index("s")
        wid = c * _NS + s
        row0 = wid * _P
        e0 = row0 * _L
        pltpu.sync_copy(z_hbm.at[pl.ds(s * _ZR, _ZR)], acc.at[pl.ds(s * _ZR, _ZR)])
        pltpu.async_copy(src_hbm.at[pl.ds(e0, _CH * _L)], src_a, isem.at[0])
        pltpu.async_copy(dst_hbm.at[pl.ds(row0, _CH)], dst_a, isem.at[0])
        plsc.subcore_barrier()

        def do_chunk(g, u):
            sv = srcb[u]
            dv = dstb[u]
            pltpu.make_async_copy(src_hbm.at[pl.ds(0, _CH * _L)], sv,
                                  isem.at[u]).wait()
            pltpu.make_async_copy(dst_hbm.at[pl.ds(0, _CH)], dv,
                                  isem.at[u]).wait()

            @pl.when(g + 1 < _NCH)
            def _():
                nxt = row0 + (g + 1) * _CH
                pltpu.async_copy(src_hbm.at[pl.ds(e0 + (g + 1) * _CH * _L,
                                                  _CH * _L)],
                                 srcb[1 - u], isem.at[1 - u])
                pltpu.async_copy(dst_hbm.at[pl.ds(nxt, _CH)], dstb[1 - u],
                                 isem.at[1 - u])

            def fire(i, b):
                pltpu.async_copy(h_hbm.at[sv.at[pl.ds(i * _L, _L)]],
                                 rows[b], gsem.at[b])

            for b in range(_NB):
                fire(b, b)
            for i in range(_CH):
                b = i % _NB
                pltpu.make_async_copy(h_hbm.at[pl.ds(0, _L)], rows[b],
                                      gsem.at[b]).wait()
                pltpu.sync_copy(rows[b], acc.at[dv.at[i]], add=True)
                if i + _NB < _CH:
                    fire(i + _NB, b)

        @pl.loop(0, _NCH // 2)
        def _(grp):
            do_chunk(grp * 2, 0)
            do_chunk(grp * 2 + 1, 1)

        plsc.subcore_barrier()
        pltpu.sync_copy(acc.at[pl.ds(s * _ZR, _ZR)],
                        out_hbm.at[c].at[pl.ds(s * _ZR, _ZR)])

    return k(h, src2d, dst2d, zeros_init)


def _tc_matmul(x, W):
    def body(x_ref, w_ref, o_ref):
        o_ref[...] = jnp.dot(x_ref[...], w_ref[...],
                             preferred_element_type=jnp.float32,
                             precision=lax.Precision.HIGHEST)

    return pl.pallas_call(
        body, out_shape=jax.ShapeDtypeStruct((_N, _D), jnp.float32))(x, W)


def _tc_prep(degp, h_raw):
    def body(dp_ref, h_ref, dinv_ref, hp_ref):
        deg = dp_ref[0, : _N, 0:1] + dp_ref[1, : _N, 0:1] + 1.0
        dinv = lax.rsqrt(deg)
        dinv_ref[...] = dinv
        hp_ref[...] = h_ref[...] * dinv

    return pl.pallas_call(
        body,
        out_shape=(jax.ShapeDtypeStruct((_N, 1), jnp.float32),
                   jax.ShapeDtypeStruct((_N, _D), jnp.float32)),
    )(degp, h_raw)


def _tc_post(p, hp, dinv, b, g, bt, W_next):
    def body(p_ref, hp_ref, dinv_ref, b_ref, g_ref, bt_ref, *rest):
        t = (p_ref[0, : _N] + p_ref[1, : _N] + hp_ref[...]) * dinv_ref[...] + b_ref[...]
        mu = jnp.mean(t, axis=0, keepdims=True)
        var = jnp.mean((t - mu) ** 2, axis=0, keepdims=True)
        y = jax.nn.relu(g_ref[...] * (t - mu) * lax.rsqrt(var + 1e-5)
                        + bt_ref[...])
        if W_next is None:
            rest[0][...] = y
        else:
            w_ref, o_ref = rest
            o_ref[...] = jnp.dot(y, w_ref[...],
                                 preferred_element_type=jnp.float32,
                                 precision=lax.Precision.HIGHEST) * dinv_ref[...]

    out_shape = jax.ShapeDtypeStruct((_N, _D), jnp.float32)
    args = (p, hp, dinv, b, g, bt)
    if W_next is not None:
        args = args + (W_next,)
    return pl.pallas_call(body, out_shape=out_shape)(*args)


def kernel(x, edge_index, W1, b1, g1, bt1, W2, b2, g2, bt2):
    src = edge_index[0]
    dst = edge_index[1]
    pad = _E_PAD - _E
    iota = jnp.arange(pad, dtype=jnp.int32)
    src1d = jnp.concatenate([src, iota % _N])
    dst2d = jnp.concatenate(
        [dst, _N + iota % (_N_PAD - _N)]).reshape(_NW * _P, _L)
    ones2d = jnp.ones((_L, _D), jnp.float32)
    zerosD = jnp.zeros((_N_PAD, _D), jnp.float32)

    degp = _sc_degree(dst2d, ones2d, zerosD)
    h1_raw = _tc_matmul(x, W1)
    dinv, h1p = _tc_prep(degp, h1_raw)
    p1 = _sc_scatter(h1p, src1d, dst2d, zerosD)
    h2p = _tc_post(p1, h1p, dinv, b1, g1, bt1, W2)
    p2 = _sc_scatter(h2p, src1d, dst2d, zerosD)
    return _tc_post(p2, h2p, dinv, b2, g2, bt2, None)
